# baseline (speedup 1.0000x reference)
import jax
import jax.numpy as jnp
from jax import lax
from jax.experimental import pallas as pl
from jax.experimental.pallas import tpu as pltpu
from jax.experimental.pallas import tpu_sc as plsc

NUM_ROWS = 1000000
NUM_FACTORS = 64
BATCH = 16384
NW = 32
BPW = BATCH // NW
LANES = 16
GROUPS = BPW // LANES
STAGE = BPW // 128
CH = 16
NCH = BPW // CH
KT = NUM_FACTORS // 8
CHUNK_BYTES = 2 * CH * KT * 8 * 16 * 4


def _fire_chunk(LT_hbm, RT_hbm, idx_u, idx_v, tu3, tv3, sems, b, c):
    base = c * CH
    r = lax.shift_right_logical(base, 7)
    o = lax.bitwise_and(base, 127)
    uvec = idx_u[r, pl.ds(o, LANES)]
    vvec = idx_v[r, pl.ds(o, LANES)]
    for i in range(CH):
        u16 = pl.multiple_of(lax.bitwise_and(uvec[i], jnp.int32(-16)), 16)
        v16 = pl.multiple_of(lax.bitwise_and(vvec[i], jnp.int32(-16)), 16)
        for kt in range(KT):
            pltpu.async_copy(
                LT_hbm.at[pl.ds(kt * 8, 8), pl.ds(u16, 16)],
                tu3.at[b, pl.ds(i * 8, 8), pl.ds(kt * 16, 16)], sems.at[b])
            pltpu.async_copy(
                RT_hbm.at[pl.ds(kt * 8, 8), pl.ds(v16, 16)],
                tv3.at[b, pl.ds(i * 8, 8), pl.ds(kt * 16, 16)], sems.at[b])


def _compute_chunk(lo4_u, lo4_v, tu3, tv3, out_vmem, lane, b, c):
    base = c * CH
    bvec = jnp.full((LANES,), 0, jnp.int32) + b
    prow = lane * 8
    cu = lo4_u[pl.ds(base, LANES)]
    cv = lo4_v[pl.ds(base, LANES)]
    acc = jnp.zeros((LANES,), jnp.float32)
    for kt in range(KT):
        cbu = cu + kt * 16
        cbv = cv + kt * 16
        for ks in range(8):
            uu = plsc.load_gather(tu3, [bvec, prow + ks, cbu])
            vv = plsc.load_gather(tv3, [bvec, prow + ks, cbv])
            acc = acc + uu * vv
    out_vmem[pl.ds(base, LANES)] = acc


def _body(users_hbm, items_hbm, LT_hbm, RT_hbm, dummy_hbm,
          out_hbm, idx_u, idx_v, lo4_u, lo4_v, tu3, tv3, out_vmem, sems):
    cid = lax.axis_index("c")
    sid = lax.axis_index("s")
    wid = sid * 2 + cid

    pltpu.sync_copy(users_hbm.at[wid], idx_u)
    pltpu.sync_copy(items_hbm.at[wid], idx_v)

    lane = lax.iota(jnp.int32, LANES)

    for j in range(GROUPS):
        r, o = divmod(j * LANES, 128)
        u = idx_u[r, pl.ds(o, LANES)]
        v = idx_v[r, pl.ds(o, LANES)]
        lo4_u[pl.ds(j * LANES, LANES)] = lax.bitwise_and(u, 15)
        lo4_v[pl.ds(j * LANES, LANES)] = lax.bitwise_and(v, 15)

    @pl.loop(0, NCH + 1)
    def _pipe(c):
        b = lax.bitwise_and(c, 1)

        @pl.when(c < NCH)
        def _fire_cur():
            _fire_chunk(LT_hbm, RT_hbm, idx_u, idx_v, tu3, tv3, sems, b, c)

        @pl.when(c > 0)
        def _compute_prev():
            pb = 1 - b
            pltpu.make_async_copy(dummy_hbm, tu3.at[pb], sems.at[pb]).wait()
            pltpu.make_async_copy(dummy_hbm, tv3.at[pb], sems.at[pb]).wait()
            _compute_chunk(lo4_u, lo4_v, tu3, tv3, out_vmem, lane, pb, c - 1)

    pltpu.sync_copy(out_vmem, out_hbm.at[pl.ds(wid * BPW, BPW)])


@jax.jit
def _mf_score(users, items, LT, RT, dummy):
    mesh = plsc.VectorSubcoreMesh(
        core_axis_name="c", subcore_axis_name="s", num_cores=2, num_subcores=16)
    kern = pl.kernel(
        _body,
        out_type=jax.ShapeDtypeStruct((BATCH,), jnp.float32),
        mesh=mesh,
        scratch_types=[
            pltpu.VMEM((STAGE, 128), jnp.int32),
            pltpu.VMEM((STAGE, 128), jnp.int32),
            pltpu.VMEM((BPW,), jnp.int32),
            pltpu.VMEM((BPW,), jnp.int32),
            pltpu.VMEM((2, CH * 8, 128), jnp.float32),
            pltpu.VMEM((2, CH * 8, 128), jnp.float32),
            pltpu.VMEM((BPW,), jnp.float32),
            pltpu.SemaphoreType.DMA((2,)),
        ],
        compiler_params=pltpu.CompilerParams(
            needs_layout_passes=False, use_tc_tiling_on_sc=True),
    )
    return kern(users, items, LT, RT, dummy)


def kernel(minibatch, L, R, L_bias, R_bias):
    users = minibatch[:, 0]
    items = minibatch[:, 1]
    bu = jnp.take(L_bias, users, axis=0)[:, 0]
    bv = jnp.take(R_bias, items, axis=0)[:, 0]
    dummy = jnp.zeros((CH * 8, 128), jnp.float32)
    dots = _mf_score(users.reshape(NW, STAGE, 128),
                     items.reshape(NW, STAGE, 128), L.T, R.T, dummy)
    return dots + bu + bv


# speedup vs baseline: 1.0043x; 1.0043x over previous
import jax
import jax.numpy as jnp
from jax import lax
from jax.experimental import pallas as pl
from jax.experimental.pallas import tpu as pltpu
from jax.experimental.pallas import tpu_sc as plsc

NUM_ROWS = 1000000
NUM_FACTORS = 64
BATCH = 16384
NW = 32
BPW = BATCH // NW
LANES = 16
GROUPS = BPW // LANES
STAGE = BPW // 128
CH = 16
NCH = BPW // CH
KT = NUM_FACTORS // 8
CHUNK_BYTES = 2 * CH * KT * 8 * 16 * 4


def _fire_chunk(LT_hbm, RT_hbm, idx_u, idx_v, tu3, tv3, sems, b, c):
    base = c * CH
    r = lax.shift_right_logical(base, 7)
    o = lax.bitwise_and(base, 127)
    uvec = idx_u[r, pl.ds(o, LANES)]
    vvec = idx_v[r, pl.ds(o, LANES)]
    for i in range(CH):
        u16 = pl.multiple_of(lax.bitwise_and(uvec[i], jnp.int32(-16)), 16)
        v16 = pl.multiple_of(lax.bitwise_and(vvec[i], jnp.int32(-16)), 16)
        for kt in range(KT):
            pltpu.async_copy(
                LT_hbm.at[pl.ds(kt * 8, 8), pl.ds(u16, 16)],
                tu3.at[b, pl.ds(i * 8, 8), pl.ds(kt * 16, 16)], sems.at[b])
            pltpu.async_copy(
                RT_hbm.at[pl.ds(kt * 8, 8), pl.ds(v16, 16)],
                tv3.at[b, pl.ds(i * 8, 8), pl.ds(kt * 16, 16)], sems.at[b])


def _compute_chunk(bias_u, bias_v, lo4_u, lo4_v, tu3, tv3, out_vmem,
                   lane, b, c):
    base = c * CH
    r = lax.shift_right_logical(base, 7)
    o = lax.bitwise_and(base, 127)
    bvec = jnp.full((LANES,), 0, jnp.int32) + b
    prow = lane * 8
    cu = lo4_u[pl.ds(base, LANES)]
    cv = lo4_v[pl.ds(base, LANES)]
    acc = bias_u[r, pl.ds(o, LANES)] + bias_v[r, pl.ds(o, LANES)]
    for kt in range(KT):
        cbu = cu + kt * 16
        cbv = cv + kt * 16
        for ks in range(8):
            uu = plsc.load_gather(tu3, [bvec, prow + ks, cbu])
            vv = plsc.load_gather(tv3, [bvec, prow + ks, cbv])
            acc = acc + uu * vv
    out_vmem[pl.ds(base, LANES)] = acc


def _body(users_hbm, items_hbm, LT_hbm, RT_hbm, bu_hbm, bv_hbm, dummy_hbm,
          out_hbm, idx_u, idx_v, bias_u, bias_v, lo4_u, lo4_v,
          tu3, tv3, out_vmem, sems):
    cid = lax.axis_index("c")
    sid = lax.axis_index("s")
    wid = sid * 2 + cid

    pltpu.sync_copy(users_hbm.at[wid], idx_u)
    pltpu.sync_copy(items_hbm.at[wid], idx_v)
    pltpu.sync_copy(bu_hbm.at[wid], bias_u)
    pltpu.sync_copy(bv_hbm.at[wid], bias_v)

    lane = lax.iota(jnp.int32, LANES)

    for j in range(GROUPS):
        r, o = divmod(j * LANES, 128)
        u = idx_u[r, pl.ds(o, LANES)]
        v = idx_v[r, pl.ds(o, LANES)]
        lo4_u[pl.ds(j * LANES, LANES)] = lax.bitwise_and(u, 15)
        lo4_v[pl.ds(j * LANES, LANES)] = lax.bitwise_and(v, 15)

    @pl.loop(0, NCH + 1)
    def _pipe(c):
        b = lax.bitwise_and(c, 1)

        @pl.when(c < NCH)
        def _fire_cur():
            _fire_chunk(LT_hbm, RT_hbm, idx_u, idx_v, tu3, tv3, sems, b, c)

        @pl.when(c > 0)
        def _compute_prev():
            pb = 1 - b
            pltpu.make_async_copy(dummy_hbm, tu3.at[pb], sems.at[pb]).wait()
            pltpu.make_async_copy(dummy_hbm, tv3.at[pb], sems.at[pb]).wait()
            _compute_chunk(bias_u, bias_v, lo4_u, lo4_v, tu3, tv3,
                           out_vmem, lane, pb, c - 1)

    pltpu.sync_copy(out_vmem, out_hbm.at[pl.ds(wid * BPW, BPW)])


@jax.jit
def _mf_score(users, items, LT, RT, bu, bv, dummy):
    mesh = plsc.VectorSubcoreMesh(
        core_axis_name="c", subcore_axis_name="s", num_cores=2, num_subcores=16)
    kern = pl.kernel(
        _body,
        out_type=jax.ShapeDtypeStruct((BATCH,), jnp.float32),
        mesh=mesh,
        scratch_types=[
            pltpu.VMEM((STAGE, 128), jnp.int32),
            pltpu.VMEM((STAGE, 128), jnp.int32),
            pltpu.VMEM((STAGE, 128), jnp.float32),
            pltpu.VMEM((STAGE, 128), jnp.float32),
            pltpu.VMEM((BPW,), jnp.int32),
            pltpu.VMEM((BPW,), jnp.int32),
            pltpu.VMEM((2, CH * 8, 128), jnp.float32),
            pltpu.VMEM((2, CH * 8, 128), jnp.float32),
            pltpu.VMEM((BPW,), jnp.float32),
            pltpu.SemaphoreType.DMA((2,)),
        ],
        compiler_params=pltpu.CompilerParams(
            needs_layout_passes=False, use_tc_tiling_on_sc=True),
    )
    return kern(users, items, LT, RT, bu, bv, dummy)


def kernel(minibatch, L, R, L_bias, R_bias):
    users = minibatch[:, 0]
    items = minibatch[:, 1]
    bu = jnp.take(L_bias, users, axis=0)[:, 0].reshape(NW, STAGE, 128)
    bv = jnp.take(R_bias, items, axis=0)[:, 0].reshape(NW, STAGE, 128)
    dummy = jnp.zeros((CH * 8, 128), jnp.float32)
    return _mf_score(users.reshape(NW, STAGE, 128),
                     items.reshape(NW, STAGE, 128),
                     L.T, R.T, bu, bv, dummy)
